# Initial kernel scaffold; baseline (speedup 1.0000x reference)
import jax
import jax.numpy as jnp
from jax.experimental import pallas as pl


def kernel(x, W, b):
    raise NotImplementedError("write your pallas kernel here")


# speedup vs baseline: 1.5652x; 1.5652x over previous
import functools

import jax
import jax.numpy as jnp
from jax import lax
from jax.experimental import pallas as pl
from jax.experimental.pallas import tpu as pltpu
from jax.experimental.pallas import tpu_sc as plsc

E = 64
K = 2
NUM_CORES = 2
NUM_SUBCORES = 16
NUM_WORKERS = NUM_CORES * NUM_SUBCORES
LANES = 16
ROW_BLK = 2048


def _logits_body(x_ref, wt_ref, b_ref, out_ref):
    out_ref[...] = (
        jnp.dot(x_ref[...], wt_ref[...], preferred_element_type=jnp.float32)
        + b_ref[...]
    )


def _tc_logits(xf, Wt, b2):
    n, d = xf.shape
    grid = n // ROW_BLK
    return pl.pallas_call(
        _logits_body,
        grid=(grid,),
        in_specs=[
            pl.BlockSpec((ROW_BLK, d), lambda i: (i, 0)),
            pl.BlockSpec((d, E), lambda i: (0, 0)),
            pl.BlockSpec((1, E), lambda i: (0, 0)),
        ],
        out_specs=pl.BlockSpec((ROW_BLK, E), lambda i: (i, 0)),
        out_shape=jax.ShapeDtypeStruct((n, E), jnp.float32),
    )(xf, Wt, b2)


def _router_body(rows_w, logits_hbm, i1_hbm, i2_hbm, w1_hbm, w2_hbm,
                 buf, i1v, i2v, w1v, w2v):
    wid = lax.axis_index("s") * NUM_CORES + lax.axis_index("c")
    base = wid * rows_w
    pltpu.sync_copy(logits_hbm.at[pl.ds(base * E, rows_w * E)], buf)

    lane = lax.iota(jnp.int32, LANES)
    lane_e = lane * E

    def group(g, carry):
        row0 = g * LANES
        flat0 = row0 * E + lane_e
        m1 = jnp.full((LANES,), -1.0, jnp.float32)
        m2 = jnp.full((LANES,), -1.0, jnp.float32)
        i1 = jnp.zeros((LANES,), jnp.int32)
        i2 = jnp.zeros((LANES,), jnp.int32)
        ssum = jnp.zeros((LANES,), jnp.float32)
        for e in range(E):
            col = jnp.full((LANES,), e, jnp.int32)
            v = plsc.load_gather(buf, [flat0 + e])
            ev = jnp.exp(v)
            ssum = ssum + ev
            is1 = ev > m1
            is2 = ev > m2
            m2, i2 = (
                jnp.where(is1, m1, jnp.where(is2, ev, m2)),
                jnp.where(is1, i1, jnp.where(is2, col, i2)),
            )
            m1 = jnp.where(is1, ev, m1)
            i1 = jnp.where(is1, col, i1)
        i1v[pl.ds(row0, LANES)] = i1
        i2v[pl.ds(row0, LANES)] = i2
        w1v[pl.ds(row0, LANES)] = m1 / ssum
        w2v[pl.ds(row0, LANES)] = m2 / ssum
        return carry

    lax.fori_loop(0, rows_w // LANES, group, 0)

    pltpu.sync_copy(i1v, i1_hbm.at[pl.ds(base, rows_w)])
    pltpu.sync_copy(i2v, i2_hbm.at[pl.ds(base, rows_w)])
    pltpu.sync_copy(w1v, w1_hbm.at[pl.ds(base, rows_w)])
    pltpu.sync_copy(w2v, w2_hbm.at[pl.ds(base, rows_w)])


def _sc_router(logits_flat):
    n = logits_flat.shape[0] // E
    rows_w = n // NUM_WORKERS
    mesh = plsc.VectorSubcoreMesh(
        core_axis_name="c", subcore_axis_name="s",
        num_cores=NUM_CORES, num_subcores=NUM_SUBCORES)
    return pl.kernel(
        functools.partial(_router_body, rows_w),
        out_type=(
            jax.ShapeDtypeStruct((n,), jnp.int32),
            jax.ShapeDtypeStruct((n,), jnp.int32),
            jax.ShapeDtypeStruct((n,), jnp.float32),
            jax.ShapeDtypeStruct((n,), jnp.float32),
        ),
        mesh=mesh,
        compiler_params=pltpu.CompilerParams(needs_layout_passes=False),
        scratch_types=[
            pltpu.VMEM((rows_w * E,), jnp.float32),
            pltpu.VMEM((rows_w,), jnp.int32),
            pltpu.VMEM((rows_w,), jnp.int32),
            pltpu.VMEM((rows_w,), jnp.float32),
            pltpu.VMEM((rows_w,), jnp.float32),
        ],
    )(logits_flat)


def kernel(x, W, b):
    bsz, seq, d = x.shape
    n = bsz * seq
    xf = x.reshape(n, d)
    logits = _tc_logits(xf, W.T, b.reshape(1, E))
    i1, i2, w1, w2 = _sc_router(logits.reshape(n * E))
    idx = jnp.stack([i1, i2], axis=-1).reshape(bsz, seq, K)
    wts = jnp.stack([w1, w2], axis=-1).reshape(bsz, seq, K)
    return idx, wts


# speedup vs baseline: 2.4685x; 1.5771x over previous
import functools

import jax
import jax.numpy as jnp
from jax import lax
from jax.experimental import pallas as pl
from jax.experimental.pallas import tpu as pltpu
from jax.experimental.pallas import tpu_sc as plsc

E = 64
K = 2
NUM_CORES = 2
NUM_SUBCORES = 16
NUM_WORKERS = NUM_CORES * NUM_SUBCORES
LANES = 16
ROW_BLK = 2048
GROUPS = 4


def _logits_body(x_ref, w_ref, b_ref, out_ref):
    acc = lax.dot_general(
        w_ref[...], x_ref[...],
        dimension_numbers=(((1,), (1,)), ((), ())),
        preferred_element_type=jnp.float32,
    )
    out_ref[...] = acc + b_ref[...]


def _tc_logits_t(xf, W, bc):
    n, d = xf.shape
    grid = n // ROW_BLK
    return pl.pallas_call(
        _logits_body,
        grid=(grid,),
        in_specs=[
            pl.BlockSpec((ROW_BLK, d), lambda i: (i, 0)),
            pl.BlockSpec((E, d), lambda i: (0, 0)),
            pl.BlockSpec((E, 1), lambda i: (0, 0)),
        ],
        out_specs=pl.BlockSpec((E, ROW_BLK), lambda i: (0, i)),
        out_shape=jax.ShapeDtypeStruct((E, n), jnp.float32),
    )(xf, W, bc)


def _router_body(rows_w, logits_hbm, i1_hbm, i2_hbm, w1_hbm, w2_hbm,
                 buf, i1v, i2v, w1v, w2v):
    wid = lax.axis_index("s") * NUM_CORES + lax.axis_index("c")
    base = wid * rows_w
    pltpu.sync_copy(logits_hbm.at[:, pl.ds(base, rows_w)], buf)

    def block(gb, carry):
        row0 = gb * (LANES * GROUPS)
        m1 = [jnp.full((LANES,), -1.0, jnp.float32) for _ in range(GROUPS)]
        m2 = [jnp.full((LANES,), -1.0, jnp.float32) for _ in range(GROUPS)]
        i1 = [jnp.zeros((LANES,), jnp.int32) for _ in range(GROUPS)]
        i2 = [jnp.zeros((LANES,), jnp.int32) for _ in range(GROUPS)]
        ssum = [jnp.zeros((LANES,), jnp.float32) for _ in range(GROUPS)]
        for e in range(E):
            col = jnp.full((LANES,), e, jnp.int32)
            for g in range(GROUPS):
                v = buf[e, pl.ds(row0 + g * LANES, LANES)]
                ev = jnp.exp(v)
                ssum[g] = ssum[g] + ev
                is1 = ev > m1[g]
                is2 = ev > m2[g]
                m2[g], i2[g] = (
                    jnp.where(is1, m1[g], jnp.where(is2, ev, m2[g])),
                    jnp.where(is1, i1[g], jnp.where(is2, col, i2[g])),
                )
                m1[g] = jnp.where(is1, ev, m1[g])
                i1[g] = jnp.where(is1, col, i1[g])
        for g in range(GROUPS):
            r0 = row0 + g * LANES
            i1v[pl.ds(r0, LANES)] = i1[g]
            i2v[pl.ds(r0, LANES)] = i2[g]
            w1v[pl.ds(r0, LANES)] = m1[g] / ssum[g]
            w2v[pl.ds(r0, LANES)] = m2[g] / ssum[g]
        return carry

    lax.fori_loop(0, rows_w // (LANES * GROUPS), block, 0)

    pltpu.sync_copy(i1v, i1_hbm.at[pl.ds(base, rows_w)])
    pltpu.sync_copy(i2v, i2_hbm.at[pl.ds(base, rows_w)])
    pltpu.sync_copy(w1v, w1_hbm.at[pl.ds(base, rows_w)])
    pltpu.sync_copy(w2v, w2_hbm.at[pl.ds(base, rows_w)])


def _sc_router(logits_t):
    n = logits_t.shape[1]
    rows_w = n // NUM_WORKERS
    mesh = plsc.VectorSubcoreMesh(
        core_axis_name="c", subcore_axis_name="s",
        num_cores=NUM_CORES, num_subcores=NUM_SUBCORES)
    return pl.kernel(
        functools.partial(_router_body, rows_w),
        out_type=(
            jax.ShapeDtypeStruct((n,), jnp.int32),
            jax.ShapeDtypeStruct((n,), jnp.int32),
            jax.ShapeDtypeStruct((n,), jnp.float32),
            jax.ShapeDtypeStruct((n,), jnp.float32),
        ),
        mesh=mesh,
        compiler_params=pltpu.CompilerParams(needs_layout_passes=False),
        scratch_types=[
            pltpu.VMEM((E, rows_w), jnp.float32),
            pltpu.VMEM((rows_w,), jnp.int32),
            pltpu.VMEM((rows_w,), jnp.int32),
            pltpu.VMEM((rows_w,), jnp.float32),
            pltpu.VMEM((rows_w,), jnp.float32),
        ],
    )(logits_t)


def kernel(x, W, b):
    bsz, seq, d = x.shape
    n = bsz * seq
    xf = x.reshape(n, d)
    logits_t = _tc_logits_t(xf, W, b.reshape(E, 1))
    i1, i2, w1, w2 = _sc_router(logits_t)
    idx = jnp.stack([i1, i2], axis=-1).reshape(bsz, seq, K)
    wts = jnp.stack([w1, w2], axis=-1).reshape(bsz, seq, K)
    return idx, wts


# speedup vs baseline: 2.5172x; 1.0197x over previous
import functools

import jax
import jax.numpy as jnp
from jax import lax
from jax.experimental import pallas as pl
from jax.experimental.pallas import tpu as pltpu
from jax.experimental.pallas import tpu_sc as plsc

E = 64
K = 2
NUM_CORES = 2
NUM_SUBCORES = 16
NUM_WORKERS = NUM_CORES * NUM_SUBCORES
LANES = 16
ROW_BLK = 4096
GROUPS = 4


def _logits_body(x_ref, w_ref, b_ref, out_ref):
    acc = lax.dot_general(
        w_ref[...], x_ref[...],
        dimension_numbers=(((1,), (1,)), ((), ())),
        preferred_element_type=jnp.float32,
    )
    out_ref[...] = acc + b_ref[...]


def _tc_logits_t(xf, W, bc):
    n, d = xf.shape
    grid = n // ROW_BLK
    return pl.pallas_call(
        _logits_body,
        grid=(grid,),
        in_specs=[
            pl.BlockSpec((ROW_BLK, d), lambda i: (i, 0)),
            pl.BlockSpec((E, d), lambda i: (0, 0)),
            pl.BlockSpec((E, 1), lambda i: (0, 0)),
        ],
        out_specs=pl.BlockSpec((E, ROW_BLK), lambda i: (0, i)),
        out_shape=jax.ShapeDtypeStruct((E, n), jnp.float32),
    )(xf, W, bc)


def _router_body(rows_w, logits_hbm, i1_hbm, i2_hbm, w1_hbm, w2_hbm,
                 buf, i1v, i2v, w1v, w2v):
    wid = lax.axis_index("s") * NUM_CORES + lax.axis_index("c")
    base = wid * rows_w
    pltpu.sync_copy(logits_hbm.at[:, pl.ds(base, rows_w)], buf)

    def block(gb, carry):
        row0 = gb * (LANES * GROUPS)
        m1 = [jnp.full((LANES,), -1.0, jnp.float32) for _ in range(GROUPS)]
        m2 = [jnp.full((LANES,), -1.0, jnp.float32) for _ in range(GROUPS)]
        i1 = [jnp.zeros((LANES,), jnp.int32) for _ in range(GROUPS)]
        i2 = [jnp.zeros((LANES,), jnp.int32) for _ in range(GROUPS)]
        ssum = [jnp.zeros((LANES,), jnp.float32) for _ in range(GROUPS)]
        for e in range(E):
            col = jnp.full((LANES,), e, jnp.int32)
            for g in range(GROUPS):
                v = buf[e, pl.ds(row0 + g * LANES, LANES)]
                ev = jnp.exp(v)
                ssum[g] = ssum[g] + ev
                is1 = ev > m1[g]
                is2 = ev > m2[g]
                m2[g], i2[g] = (
                    jnp.where(is1, m1[g], jnp.where(is2, ev, m2[g])),
                    jnp.where(is1, i1[g], jnp.where(is2, col, i2[g])),
                )
                m1[g] = jnp.where(is1, ev, m1[g])
                i1[g] = jnp.where(is1, col, i1[g])
        for g in range(GROUPS):
            r0 = row0 + g * LANES
            i1v[pl.ds(r0, LANES)] = i1[g]
            i2v[pl.ds(r0, LANES)] = i2[g]
            w1v[pl.ds(r0, LANES)] = m1[g] / ssum[g]
            w2v[pl.ds(r0, LANES)] = m2[g] / ssum[g]
        return carry

    lax.fori_loop(0, rows_w // (LANES * GROUPS), block, 0)

    pltpu.sync_copy(i1v, i1_hbm.at[pl.ds(base, rows_w)])
    pltpu.sync_copy(i2v, i2_hbm.at[pl.ds(base, rows_w)])
    pltpu.sync_copy(w1v, w1_hbm.at[pl.ds(base, rows_w)])
    pltpu.sync_copy(w2v, w2_hbm.at[pl.ds(base, rows_w)])


def _sc_router(logits_t):
    n = logits_t.shape[1]
    rows_w = n // NUM_WORKERS
    mesh = plsc.VectorSubcoreMesh(
        core_axis_name="c", subcore_axis_name="s",
        num_cores=NUM_CORES, num_subcores=NUM_SUBCORES)
    return pl.kernel(
        functools.partial(_router_body, rows_w),
        out_type=(
            jax.ShapeDtypeStruct((n,), jnp.int32),
            jax.ShapeDtypeStruct((n,), jnp.int32),
            jax.ShapeDtypeStruct((n,), jnp.float32),
            jax.ShapeDtypeStruct((n,), jnp.float32),
        ),
        mesh=mesh,
        compiler_params=pltpu.CompilerParams(needs_layout_passes=False),
        scratch_types=[
            pltpu.VMEM((E, rows_w), jnp.float32),
            pltpu.VMEM((rows_w,), jnp.int32),
            pltpu.VMEM((rows_w,), jnp.int32),
            pltpu.VMEM((rows_w,), jnp.float32),
            pltpu.VMEM((rows_w,), jnp.float32),
        ],
    )(logits_t)


def kernel(x, W, b):
    bsz, seq, d = x.shape
    n = bsz * seq
    xf = x.reshape(n, d)
    logits_t = _tc_logits_t(xf, W, b.reshape(E, 1))
    i1, i2, w1, w2 = _sc_router(logits_t)
    idx = jnp.stack([i1, i2], axis=-1).reshape(bsz, seq, K)
    wts = jnp.stack([w1, w2], axis=-1).reshape(bsz, seq, K)
    return idx, wts
